# Initial kernel scaffold; baseline (speedup 1.0000x reference)
import jax
import jax.numpy as jnp
from jax.experimental import pallas as pl


def kernel(x, memory_keys, memory_values, Wq, Wout, g1_w, g1_b, g2_w, g2_b):
    raise NotImplementedError("write your pallas kernel here")


# speedup vs baseline: 16.3007x; 16.3007x over previous
import functools
import math

import jax
import jax.numpy as jnp
from jax import lax
from jax.experimental import pallas as pl
from jax.experimental.pallas import tpu as pltpu
from jax.experimental.pallas import tpu_sc as plsc

TOPK = 8
LANES = 16


def _stage_a_body(x_ref, wq_ref, keys_ref, w_ref, idx_ref, sim_ref):
    R = x_ref.shape[0]
    M = keys_ref.shape[0]
    D = x_ref.shape[1]
    q = lax.dot_general(x_ref[...], wq_ref[...], (((1,), (1,)), ((), ())),
                        preferred_element_type=jnp.float32)
    sim = lax.dot_general(q, keys_ref[...], (((1,), (1,)), ((), ())),
                          preferred_element_type=jnp.float32)
    sim_ref[...] = sim * (1.0 / math.sqrt(D))

    iota = lax.broadcasted_iota(jnp.int32, (R, M), 1)
    vals = []
    idxs = []
    for k in range(TOPK):
        s = sim_ref[...]
        m = jnp.max(s, axis=1, keepdims=True)
        cand = jnp.where(s == m, iota, M)
        ik = jnp.min(cand, axis=1, keepdims=True)
        vals.append(m)
        idxs.append(ik)
        if k < TOPK - 1:
            sim_ref[...] = jnp.where(iota == ik, -jnp.inf, s)
    v = jnp.concatenate(vals, axis=1)
    ii = jnp.concatenate(idxs, axis=1)
    e = jnp.exp(v - v[:, 0:1])
    w_ref[...] = e / jnp.sum(e, axis=1, keepdims=True)
    idx_ref[...] = ii


def _make_stage_a(N, D, M, R):
    grid = (N // R,)
    return pl.pallas_call(
        _stage_a_body,
        grid=grid,
        in_specs=[
            pl.BlockSpec((R, D), lambda i: (i, 0)),
            pl.BlockSpec((D, D), lambda i: (0, 0)),
            pl.BlockSpec((M, D), lambda i: (0, 0)),
        ],
        out_specs=[
            pl.BlockSpec((R, TOPK), lambda i: (i, 0)),
            pl.BlockSpec((R, TOPK), lambda i: (i, 0)),
        ],
        out_shape=[
            jax.ShapeDtypeStruct((N, TOPK), jnp.float32),
            jax.ShapeDtypeStruct((N, TOPK), jnp.int32),
        ],
        scratch_shapes=[pltpu.VMEM((R, M), jnp.float32)],
    )


_CR = 8


def _stage_b_body(nw, rows_per_w, D, wb_hbm, idx_hbm, mv_hbm, out_hbm,
                  w_v, idx_v, rows_v, out_v, sem):
    wid = lax.axis_index("s") * 2 + lax.axis_index("c")
    n_chunks = rows_per_w // _CR
    n_d = D // LANES

    def chunk(c, carry):
        base = wid * rows_per_w + c * _CR
        pltpu.sync_copy(idx_hbm.at[pl.ds(base * TOPK, _CR * TOPK)], idx_v)
        pltpu.sync_copy(wb_hbm.at[pl.ds(base, _CR)], w_v)
        pltpu.async_copy(mv_hbm.at[idx_v], rows_v, sem).wait()
        for r in range(_CR):
            def dbody(d, _, r=r):
                off = d * LANES
                acc = w_v[r, 0, :] * rows_v[r * TOPK, pl.ds(off, LANES)]
                for k in range(1, TOPK):
                    acc = acc + (w_v[r, k, :]
                                 * rows_v[r * TOPK + k, pl.ds(off, LANES)])
                out_v[r, pl.ds(off, LANES)] = acc
                return 0
            lax.fori_loop(0, n_d, dbody, 0)
        pltpu.sync_copy(out_v, out_hbm.at[pl.ds(base, _CR)])
        return carry

    lax.fori_loop(0, n_chunks, chunk, 0)


def _make_stage_b(N, D, M):
    nw = 32
    rows_per_w = N // nw
    mesh = plsc.VectorSubcoreMesh(core_axis_name="c", subcore_axis_name="s")
    return pl.kernel(
        functools.partial(_stage_b_body, nw, rows_per_w, D),
        out_type=jax.ShapeDtypeStruct((N, D), jnp.float32),
        mesh=mesh,
        scratch_types=[
            pltpu.VMEM((_CR, TOPK, LANES), jnp.float32),
            pltpu.VMEM((_CR * TOPK,), jnp.int32),
            pltpu.VMEM((_CR * TOPK, D), jnp.float32),
            pltpu.VMEM((_CR, D), jnp.float32),
            pltpu.SemaphoreType.DMA,
        ],
    )


def _stage_c_body(x_ref, rr_ref, wout_ref, g1x_ref, g1r_ref, g1b_ref,
                  g2w_ref, g2b_ref, out_ref):
    x = x_ref[...]
    ret = lax.dot_general(rr_ref[...], wout_ref[...], (((1,), (1,)), ((), ())),
                          preferred_element_type=jnp.float32)
    h = (lax.dot_general(x, g1x_ref[...], (((1,), (1,)), ((), ())),
                         preferred_element_type=jnp.float32)
         + lax.dot_general(ret, g1r_ref[...], (((1,), (1,)), ((), ())),
                           preferred_element_type=jnp.float32)
         + g1b_ref[...])
    h = 0.5 * h * (1.0 + lax.erf(h * (1.0 / math.sqrt(2.0))))
    gpre = jnp.sum(h * g2w_ref[...], axis=1, keepdims=True) + g2b_ref[...]
    gate = jax.nn.sigmoid(gpre)
    out_ref[...] = x + gate * ret


def _make_stage_c(N, D, R):
    H = D // 2
    return pl.pallas_call(
        _stage_c_body,
        grid=(N // R,),
        in_specs=[
            pl.BlockSpec((R, D), lambda i: (i, 0)),
            pl.BlockSpec((R, D), lambda i: (i, 0)),
            pl.BlockSpec((D, D), lambda i: (0, 0)),
            pl.BlockSpec((H, D), lambda i: (0, 0)),
            pl.BlockSpec((H, D), lambda i: (0, 0)),
            pl.BlockSpec((1, H), lambda i: (0, 0)),
            pl.BlockSpec((1, H), lambda i: (0, 0)),
            pl.BlockSpec((1, 1), lambda i: (0, 0)),
        ],
        out_specs=pl.BlockSpec((R, D), lambda i: (i, 0)),
        out_shape=jax.ShapeDtypeStruct((N, D), jnp.float32),
    )


def kernel(x, memory_keys, memory_values, Wq, Wout, g1_w, g1_b, g2_w, g2_b):
    B, T, D = x.shape
    M = memory_keys.shape[0]
    N = B * T
    R = 256

    xf = x.reshape(N, D)
    w, idx = _make_stage_a(N, D, M, R)(xf, Wq, memory_keys)

    wb = jnp.broadcast_to(w[:, :, None], (N, TOPK, LANES))
    idxf = idx.reshape(N * TOPK)
    ret_raw = _make_stage_b(N, D, M)(wb, idxf, memory_values)

    H = D // 2
    out = _make_stage_c(N, D, R)(
        xf, ret_raw, Wout,
        g1_w[:, :D], g1_w[:, D:], g1_b.reshape(1, H),
        g2_w.reshape(1, H), g2_b.reshape(1, 1),
    )
    return (out.reshape(B, T, D),
            w.reshape(B, T, TOPK),
            idx.reshape(B, T, TOPK))


# speedup vs baseline: 21.3063x; 1.3071x over previous
import functools
import math

import jax
import jax.numpy as jnp
from jax import lax
from jax.experimental import pallas as pl
from jax.experimental.pallas import tpu as pltpu
from jax.experimental.pallas import tpu_sc as plsc

TOPK = 8
LANES = 16


def _stage_a_body(x_ref, wq_ref, keys_ref, w_ref, idx_ref, sim_ref):
    R = x_ref.shape[0]
    M = keys_ref.shape[0]
    D = x_ref.shape[1]
    q = lax.dot_general(x_ref[...], wq_ref[...], (((1,), (1,)), ((), ())),
                        preferred_element_type=jnp.float32)
    sim = lax.dot_general(q, keys_ref[...], (((1,), (1,)), ((), ())),
                          preferred_element_type=jnp.float32)
    sim_ref[...] = sim * (1.0 / math.sqrt(D))

    iota = lax.broadcasted_iota(jnp.int32, (R, M), 1).astype(jnp.float32)
    vals = []
    idxs = []
    for k in range(TOPK):
        s = sim_ref[...]
        m = jnp.max(s, axis=1, keepdims=True)
        cand = jnp.where(s == m, iota, float(M))
        ik = jnp.min(cand, axis=1, keepdims=True)
        vals.append(m)
        idxs.append(ik)
        if k < TOPK - 1:
            sim_ref[...] = jnp.where(iota == ik, -jnp.inf, s)
    v = jnp.concatenate(vals, axis=1)
    ii = jnp.concatenate(idxs, axis=1)
    e = jnp.exp(v - v[:, 0:1])
    w_ref[...] = e / jnp.sum(e, axis=1, keepdims=True)
    idx_ref[...] = ii.astype(jnp.int32)


def _make_stage_a(N, D, M, R):
    grid = (N // R,)
    return pl.pallas_call(
        _stage_a_body,
        grid=grid,
        in_specs=[
            pl.BlockSpec((R, D), lambda i: (i, 0)),
            pl.BlockSpec((D, D), lambda i: (0, 0)),
            pl.BlockSpec((M, D), lambda i: (0, 0)),
        ],
        out_specs=[
            pl.BlockSpec((R, TOPK), lambda i: (i, 0)),
            pl.BlockSpec((R, TOPK), lambda i: (i, 0)),
        ],
        out_shape=[
            jax.ShapeDtypeStruct((N, TOPK), jnp.float32),
            jax.ShapeDtypeStruct((N, TOPK), jnp.int32),
        ],
        scratch_shapes=[pltpu.VMEM((R, M), jnp.float32)],
    )


_CR = 4


def _stage_b_body(rows_per_w, D, wb_hbm, idx_hbm, mv_hbm, out_hbm,
                  w_v, idx_v, rows_v0, rows_v1, out_v0, out_v1,
                  gsem0, gsem1, wsem0, wsem1):
    wid = lax.axis_index("s") * 2 + lax.axis_index("c")
    n_chunks = rows_per_w // _CR
    n_d = D // LANES
    base_w = wid * rows_per_w
    rows_bufs = (rows_v0, rows_v1)
    out_bufs = (out_v0, out_v1)
    gsems = (gsem0, gsem1)
    wsems = (wsem0, wsem1)

    pltpu.sync_copy(idx_hbm.at[pl.ds(base_w * TOPK, rows_per_w * TOPK)],
                    idx_v)
    pltpu.sync_copy(wb_hbm.at[pl.ds(base_w * TOPK, rows_per_w * TOPK)],
                    w_v.at[pl.ds(0, rows_per_w * TOPK)])

    def gather(c, b):
        return pltpu.async_copy(
            mv_hbm.at[idx_v.at[pl.ds(c * (_CR * TOPK), _CR * TOPK)]],
            rows_bufs[b], gsems[b])

    gather(0, 0)
    gather(1, 1)

    def outer(cc, carry):
        for b in range(2):
            c = cc * 2 + b
            rows_b = rows_bufs[b]
            out_b = out_bufs[b]
            pltpu.make_async_copy(
                mv_hbm.at[idx_v.at[pl.ds(c * (_CR * TOPK), _CR * TOPK)]],
                rows_b, gsems[b]).wait()

            @pl.when(cc >= 1)
            def _():
                pltpu.make_async_copy(
                    out_b, out_hbm.at[pl.ds(base_w + (c - 2) * _CR, _CR)],
                    wsems[b]).wait()
            for r in range(_CR):
                row = c * _CR + r
                wrow = w_v[pl.ds(row * TOPK, LANES)]
                wv = [wrow[k] for k in range(TOPK)]

                def dbody(d, _, r=r, wv=wv, rows_b=rows_b, out_b=out_b):
                    off = d * LANES
                    acc = wv[0] * rows_b[r * TOPK, pl.ds(off, LANES)]
                    for k in range(1, TOPK):
                        acc = acc + (wv[k]
                                     * rows_b[r * TOPK + k, pl.ds(off, LANES)])
                    out_b[r, pl.ds(off, LANES)] = acc
                    return 0
                lax.fori_loop(0, n_d, dbody, 0)

            pltpu.async_copy(out_b, out_hbm.at[pl.ds(base_w + c * _CR, _CR)],
                             wsems[b])

            @pl.when(cc < n_chunks // 2 - 1)
            def _():
                gather(c + 2, b)
        return carry

    lax.fori_loop(0, n_chunks // 2, outer, 0)
    for b in range(2):
        c_last = n_chunks - 2 + b
        pltpu.make_async_copy(
            out_bufs[b], out_hbm.at[pl.ds(base_w + c_last * _CR, _CR)],
            wsems[b]).wait()


def _make_stage_b(N, D, M):
    nw = 32
    rows_per_w = N // nw
    mesh = plsc.VectorSubcoreMesh(core_axis_name="c", subcore_axis_name="s")
    return pl.kernel(
        functools.partial(_stage_b_body, rows_per_w, D),
        out_type=jax.ShapeDtypeStruct((N, D), jnp.float32),
        mesh=mesh,
        scratch_types=[
            pltpu.VMEM((rows_per_w * TOPK + LANES,), jnp.float32),
            pltpu.VMEM((rows_per_w * TOPK,), jnp.int32),
            pltpu.VMEM((_CR * TOPK, D), jnp.float32),
            pltpu.VMEM((_CR * TOPK, D), jnp.float32),
            pltpu.VMEM((_CR, D), jnp.float32),
            pltpu.VMEM((_CR, D), jnp.float32),
            pltpu.SemaphoreType.DMA,
            pltpu.SemaphoreType.DMA,
            pltpu.SemaphoreType.DMA,
            pltpu.SemaphoreType.DMA,
        ],
    )


def _stage_c_body(x_ref, rr_ref, wout_ref, g1x_ref, g1r_ref, g1b_ref,
                  g2w_ref, g2b_ref, out_ref):
    x = x_ref[...]
    ret = lax.dot_general(rr_ref[...], wout_ref[...], (((1,), (1,)), ((), ())),
                          preferred_element_type=jnp.float32)
    h = (lax.dot_general(x, g1x_ref[...], (((1,), (1,)), ((), ())),
                         preferred_element_type=jnp.float32)
         + lax.dot_general(ret, g1r_ref[...], (((1,), (1,)), ((), ())),
                           preferred_element_type=jnp.float32)
         + g1b_ref[...])
    h = 0.5 * h * (1.0 + lax.erf(h * (1.0 / math.sqrt(2.0))))
    gpre = jnp.sum(h * g2w_ref[...], axis=1, keepdims=True) + g2b_ref[...]
    gate = jax.nn.sigmoid(gpre)
    out_ref[...] = x + gate * ret


def _make_stage_c(N, D, R):
    H = D // 2
    return pl.pallas_call(
        _stage_c_body,
        grid=(N // R,),
        in_specs=[
            pl.BlockSpec((R, D), lambda i: (i, 0)),
            pl.BlockSpec((R, D), lambda i: (i, 0)),
            pl.BlockSpec((D, D), lambda i: (0, 0)),
            pl.BlockSpec((H, D), lambda i: (0, 0)),
            pl.BlockSpec((H, D), lambda i: (0, 0)),
            pl.BlockSpec((1, H), lambda i: (0, 0)),
            pl.BlockSpec((1, H), lambda i: (0, 0)),
            pl.BlockSpec((1, 1), lambda i: (0, 0)),
        ],
        out_specs=pl.BlockSpec((R, D), lambda i: (i, 0)),
        out_shape=jax.ShapeDtypeStruct((N, D), jnp.float32),
    )


def kernel(x, memory_keys, memory_values, Wq, Wout, g1_w, g1_b, g2_w, g2_b):
    B, T, D = x.shape
    M = memory_keys.shape[0]
    N = B * T
    R = 256

    xf = x.reshape(N, D)
    w, idx = _make_stage_a(N, D, M, R)(xf, Wq, memory_keys)

    wf = w.reshape(N * TOPK)
    idxf = idx.reshape(N * TOPK)
    ret_raw = _make_stage_b(N, D, M)(wf, idxf, memory_values)

    H = D // 2
    out = _make_stage_c(N, D, R)(
        xf, ret_raw, Wout,
        g1_w[:, :D], g1_w[:, D:], g1_b.reshape(1, H),
        g2_w.reshape(1, H), g2_b.reshape(1, 1),
    )
    return (out.reshape(B, T, D),
            w.reshape(B, T, TOPK),
            idx.reshape(B, T, TOPK))


# speedup vs baseline: 22.7260x; 1.0666x over previous
import functools
import math

import jax
import jax.numpy as jnp
from jax import lax
from jax.experimental import pallas as pl
from jax.experimental.pallas import tpu as pltpu
from jax.experimental.pallas import tpu_sc as plsc

TOPK = 8
LANES = 16


def _stage_a_body(x_ref, wq_ref, keys_ref, w_ref, idx_ref, sim_ref):
    R = x_ref.shape[0]
    M = keys_ref.shape[0]
    D = x_ref.shape[1]
    q = lax.dot_general(x_ref[...], wq_ref[...], (((1,), (1,)), ((), ())),
                        preferred_element_type=jnp.float32)
    sim = lax.dot_general(q, keys_ref[...], (((1,), (1,)), ((), ())),
                          preferred_element_type=jnp.float32)
    sim_ref[...] = sim * (1.0 / math.sqrt(D))

    iota = lax.broadcasted_iota(jnp.int32, (R, M), 1).astype(jnp.float32)
    vals = []
    idxs = []
    for k in range(TOPK):
        s = sim_ref[...]
        m = jnp.max(s, axis=1, keepdims=True)
        cand = jnp.where(s == m, iota, float(M))
        ik = jnp.min(cand, axis=1, keepdims=True)
        vals.append(m)
        idxs.append(ik)
        if k < TOPK - 1:
            sim_ref[...] = jnp.where(iota == ik, -jnp.inf, s)
    v = jnp.concatenate(vals, axis=1)
    ii = jnp.concatenate(idxs, axis=1)
    e = jnp.exp(v - v[:, 0:1])
    w_ref[...] = e / jnp.sum(e, axis=1, keepdims=True)
    idx_ref[...] = ii.astype(jnp.int32)


def _make_stage_a(N, D, M, R):
    grid = (N // R,)
    return pl.pallas_call(
        _stage_a_body,
        grid=grid,
        in_specs=[
            pl.BlockSpec((R, D), lambda i: (i, 0)),
            pl.BlockSpec((D, D), lambda i: (0, 0)),
            pl.BlockSpec((M, D), lambda i: (0, 0)),
        ],
        out_specs=[
            pl.BlockSpec((R, TOPK), lambda i: (i, 0)),
            pl.BlockSpec((R, TOPK), lambda i: (i, 0)),
        ],
        out_shape=[
            jax.ShapeDtypeStruct((N, TOPK), jnp.float32),
            jax.ShapeDtypeStruct((N, TOPK), jnp.int32),
        ],
        scratch_shapes=[pltpu.VMEM((R, M), jnp.float32)],
    )


_CR = 4


def _stage_b_body(rows_per_w, D, wb_hbm, idx_hbm, mv_hbm, out_hbm,
                  w_v, idx_v, rows_v0, rows_v1, out_v0, out_v1,
                  gsem0, gsem1, wsem0, wsem1):
    wid = lax.axis_index("s") * 2 + lax.axis_index("c")
    n_chunks = rows_per_w // _CR
    n_d = D // LANES
    base_w = wid * rows_per_w
    rows_bufs = (rows_v0, rows_v1)
    out_bufs = (out_v0, out_v1)
    gsems = (gsem0, gsem1)
    wsems = (wsem0, wsem1)

    pltpu.sync_copy(idx_hbm.at[pl.ds(base_w * TOPK, rows_per_w * TOPK)],
                    idx_v)
    pltpu.sync_copy(wb_hbm.at[pl.ds(base_w * TOPK, rows_per_w * TOPK)],
                    w_v.at[pl.ds(0, rows_per_w * TOPK)])

    def gather(c, b):
        return pltpu.async_copy(
            mv_hbm.at[idx_v.at[pl.ds(c * (_CR * TOPK), _CR * TOPK)]],
            rows_bufs[b], gsems[b])

    gather(0, 0)
    gather(1, 1)

    def outer(cc, carry):
        for b in range(2):
            c = cc * 2 + b
            rows_b = rows_bufs[b]
            out_b = out_bufs[b]
            pltpu.make_async_copy(
                mv_hbm.at[idx_v.at[pl.ds(c * (_CR * TOPK), _CR * TOPK)]],
                rows_b, gsems[b]).wait()

            @pl.when(cc >= 1)
            def _():
                pltpu.make_async_copy(
                    out_b, out_hbm.at[pl.ds(base_w + (c - 2) * _CR, _CR)],
                    wsems[b]).wait()
            for r in range(_CR):
                row = c * _CR + r
                wrow = w_v[pl.ds(row * TOPK, LANES)]
                wv = [wrow[k] for k in range(TOPK)]

                def dbody(d, _, r=r, wv=wv, rows_b=rows_b, out_b=out_b):
                    off = d * LANES
                    acc = wv[0] * rows_b[r * TOPK, pl.ds(off, LANES)]
                    for k in range(1, TOPK):
                        acc = acc + (wv[k]
                                     * rows_b[r * TOPK + k, pl.ds(off, LANES)])
                    out_b[r, pl.ds(off, LANES)] = acc
                    return 0
                lax.fori_loop(0, n_d, dbody, 0)

            pltpu.async_copy(out_b, out_hbm.at[pl.ds(base_w + c * _CR, _CR)],
                             wsems[b])

            @pl.when(cc < n_chunks // 2 - 1)
            def _():
                gather(c + 2, b)
        return carry

    lax.fori_loop(0, n_chunks // 2, outer, 0)
    for b in range(2):
        c_last = n_chunks - 2 + b
        pltpu.make_async_copy(
            out_bufs[b], out_hbm.at[pl.ds(base_w + c_last * _CR, _CR)],
            wsems[b]).wait()


def _make_stage_b(N, D, M):
    nw = 32
    rows_per_w = N // nw
    mesh = plsc.VectorSubcoreMesh(core_axis_name="c", subcore_axis_name="s")
    return pl.kernel(
        functools.partial(_stage_b_body, rows_per_w, D),
        out_type=jax.ShapeDtypeStruct((N, D), jnp.float32),
        mesh=mesh,
        scratch_types=[
            pltpu.VMEM((rows_per_w * TOPK + LANES,), jnp.float32),
            pltpu.VMEM((rows_per_w * TOPK,), jnp.int32),
            pltpu.VMEM((_CR * TOPK, D), jnp.float32),
            pltpu.VMEM((_CR * TOPK, D), jnp.float32),
            pltpu.VMEM((_CR, D), jnp.float32),
            pltpu.VMEM((_CR, D), jnp.float32),
            pltpu.SemaphoreType.DMA,
            pltpu.SemaphoreType.DMA,
            pltpu.SemaphoreType.DMA,
            pltpu.SemaphoreType.DMA,
        ],
    )


def _stage_c_body(x_ref, rr_ref, wout_ref, g1x_ref, g1r_ref, g1b_ref,
                  g2w_ref, g2b_ref, out_ref):
    x = x_ref[...]
    ret = lax.dot_general(rr_ref[...], wout_ref[...], (((1,), (1,)), ((), ())),
                          preferred_element_type=jnp.float32)
    h = (lax.dot_general(x, g1x_ref[...], (((1,), (1,)), ((), ())),
                         preferred_element_type=jnp.float32)
         + lax.dot_general(ret, g1r_ref[...], (((1,), (1,)), ((), ())),
                           preferred_element_type=jnp.float32)
         + g1b_ref[...])
    h = 0.5 * h * (1.0 + lax.erf(h * (1.0 / math.sqrt(2.0))))
    gpre = jnp.sum(h * g2w_ref[...], axis=1, keepdims=True) + g2b_ref[...]
    gate = jax.nn.sigmoid(gpre)
    out_ref[...] = x + gate * ret


def _make_stage_c(N, D, R):
    H = D // 2
    return pl.pallas_call(
        _stage_c_body,
        grid=(N // R,),
        in_specs=[
            pl.BlockSpec((R, D), lambda i: (i, 0)),
            pl.BlockSpec((R, D), lambda i: (i, 0)),
            pl.BlockSpec((D, D), lambda i: (0, 0)),
            pl.BlockSpec((H, D), lambda i: (0, 0)),
            pl.BlockSpec((H, D), lambda i: (0, 0)),
            pl.BlockSpec((1, H), lambda i: (0, 0)),
            pl.BlockSpec((1, H), lambda i: (0, 0)),
            pl.BlockSpec((1, 1), lambda i: (0, 0)),
        ],
        out_specs=pl.BlockSpec((R, D), lambda i: (i, 0)),
        out_shape=jax.ShapeDtypeStruct((N, D), jnp.float32),
    )


def kernel(x, memory_keys, memory_values, Wq, Wout, g1_w, g1_b, g2_w, g2_b):
    B, T, D = x.shape
    M = memory_keys.shape[0]
    N = B * T
    R = 256

    xf = x.reshape(N, D)
    H = D // 2

    NH = N // 2
    stage_a = _make_stage_a(NH, D, M, R)
    stage_b = _make_stage_b(NH, D, M)
    stage_c = _make_stage_c(NH, D, R)

    ws, idxs, outs = [], [], []
    for h in range(2):
        xh = xf[h * NH:(h + 1) * NH]
        w, idx = stage_a(xh, Wq, memory_keys)
        ret_raw = stage_b(w.reshape(NH * TOPK), idx.reshape(NH * TOPK),
                          memory_values)
        out = stage_c(
            xh, ret_raw, Wout,
            g1_w[:, :D], g1_w[:, D:], g1_b.reshape(1, H),
            g2_w.reshape(1, H), g2_b.reshape(1, 1),
        )
        ws.append(w)
        idxs.append(idx)
        outs.append(out)

    return (jnp.concatenate(outs).reshape(B, T, D),
            jnp.concatenate(ws).reshape(B, T, TOPK),
            jnp.concatenate(idxs).reshape(B, T, TOPK))
